# Initial kernel scaffold; baseline (speedup 1.0000x reference)
import jax
import jax.numpy as jnp
from jax.experimental import pallas as pl


def kernel(x, edge_index, W1, b1, W2, b2):
    raise NotImplementedError("write your pallas kernel here")


# speedup vs baseline: 7.6763x; 7.6763x over previous
import functools

import jax
import jax.numpy as jnp
from jax import lax
from jax.experimental import pallas as pl
from jax.experimental.pallas import tpu as pltpu
from jax.experimental.pallas import tpu_sc as plsc

N = 10000
E = 160000
D_IN = 256
D_H = 256
D_OUT = 64

CHUNK = 128
E_PAD = 163840
ACC_ROWS = 10240
NC, NS = 2, 16
ROWS_PER_TILE_OUT = N // NS

_mesh = plsc.VectorSubcoreMesh(core_axis_name="c", subcore_axis_name="s")


def _zero_fill(buf, ncols):
    z = jnp.zeros((16,), jnp.float32)

    @pl.loop(0, 128)
    def _(i):
        for k in range(ncols // 16):
            buf[i, pl.ds(k * 16, 16)] = z


def _deg_body(dst_hbm, deg_hbm, ones_v, dst_i, acc):
    c = lax.axis_index("c")
    s = lax.axis_index("s")
    wid = c * NS + s

    _zero_fill(ones_v, 128)
    for k in range(ACC_ROWS // NS // 128):
        pltpu.sync_copy(ones_v, acc.at[pl.ds(s * (ACC_ROWS // NS) + k * 128, 128)])

    one = jnp.ones((16,), jnp.float32)

    @pl.loop(0, 128)
    def _(i):
        for k in range(8):
            ones_v[i, pl.ds(k * 16, 16)] = one

    n_chunks = E_PAD // (NC * NS) // CHUNK
    pltpu.sync_copy(dst_hbm.at[wid], dst_i)
    plsc.subcore_barrier()

    @pl.loop(0, n_chunks)
    def _(j):
        pltpu.sync_copy(ones_v, acc.at[dst_i.at[j]], add=True)

    plsc.subcore_barrier()
    stripe = ACC_ROWS // NS
    pltpu.sync_copy(acc.at[pl.ds(s * stripe, stripe)],
                    deg_hbm.at[pl.ds(c * ACC_ROWS + s * stripe, stripe)])


_deg_call = pl.kernel(
    _deg_body,
    out_type=jax.ShapeDtypeStruct((NC * ACC_ROWS, 128), jnp.float32),
    mesh=_mesh,
    scratch_types=[
        pltpu.VMEM((CHUNK, 128), jnp.float32),
        pltpu.VMEM((E_PAD // (NC * NS) // CHUNK, CHUNK), jnp.int32),
        pltpu.VMEM_SHARED((ACC_ROWS, 128), jnp.float32),
    ],
)


def _agg256_body(srcs_hbm, dst_hbm, tab_hbm, out_hbm, src_i, dst_i,
                 rows_v, acc):
    c = lax.axis_index("c")
    s = lax.axis_index("s")

    _zero_fill(rows_v, 128)
    for k in range(ACC_ROWS // NS // 128):
        pltpu.sync_copy(rows_v, acc.at[pl.ds(s * (ACC_ROWS // NS) + k * 128, 128)])

    n_chunks = E_PAD // NS // CHUNK
    pltpu.sync_copy(srcs_hbm.at[c * NS + s], src_i)
    pltpu.sync_copy(dst_hbm.at[s], dst_i)
    plsc.subcore_barrier()

    @pl.loop(0, n_chunks)
    def _(j):
        pltpu.sync_copy(tab_hbm.at[src_i.at[j]], rows_v)
        pltpu.sync_copy(rows_v, acc.at[dst_i.at[j]], add=True)

    plsc.subcore_barrier()
    stripe = ACC_ROWS // NS
    pltpu.sync_copy(acc.at[pl.ds(s * stripe, stripe)],
                    out_hbm.at[pl.ds(c * ACC_ROWS + s * stripe, stripe)])


_agg256_call = pl.kernel(
    _agg256_body,
    out_type=jax.ShapeDtypeStruct((NC * ACC_ROWS, 128), jnp.float32),
    mesh=_mesh,
    scratch_types=[
        pltpu.VMEM((E_PAD // NS // CHUNK, CHUNK), jnp.int32),
        pltpu.VMEM((E_PAD // NS // CHUNK, CHUNK), jnp.int32),
        pltpu.VMEM((CHUNK, 128), jnp.float32),
        pltpu.VMEM_SHARED((ACC_ROWS, 128), jnp.float32),
    ],
)


def _agg64_body(src_hbm, dst_hbm, tab_hbm, out_hbm, src_i, dst_i,
                rows_v, acc):
    c = lax.axis_index("c")
    s = lax.axis_index("s")
    wid = c * NS + s

    _zero_fill(rows_v, 128)
    for k in range(ACC_ROWS // NS // 128):
        pltpu.sync_copy(rows_v, acc.at[pl.ds(s * (ACC_ROWS // NS) + k * 128, 128)])

    n_chunks = E_PAD // (NC * NS) // CHUNK
    pltpu.sync_copy(src_hbm.at[wid], src_i)
    pltpu.sync_copy(dst_hbm.at[wid], dst_i)
    plsc.subcore_barrier()

    @pl.loop(0, n_chunks)
    def _(j):
        pltpu.sync_copy(tab_hbm.at[src_i.at[j]], rows_v)
        pltpu.sync_copy(rows_v, acc.at[dst_i.at[j]], add=True)

    plsc.subcore_barrier()
    stripe = ACC_ROWS // NS
    pltpu.sync_copy(acc.at[pl.ds(s * stripe, stripe)],
                    out_hbm.at[pl.ds(c * ACC_ROWS + s * stripe, stripe)])


_agg64_call = pl.kernel(
    _agg64_body,
    out_type=jax.ShapeDtypeStruct((NC * ACC_ROWS, 128), jnp.float32),
    mesh=_mesh,
    scratch_types=[
        pltpu.VMEM((E_PAD // (NC * NS) // CHUNK, CHUNK), jnp.int32),
        pltpu.VMEM((E_PAD // (NC * NS) // CHUNK, CHUNK), jnp.int32),
        pltpu.VMEM((CHUNK, 128), jnp.float32),
        pltpu.VMEM_SHARED((ACC_ROWS, 128), jnp.float32),
    ],
)


BN = 1000


def _dinv_from_parts(degp):
    deg = degp[0, :, 0:1] + degp[1, :, 0:1] + 1.0
    return lax.rsqrt(deg)


def _tc_a_body(x_ref, w1_ref, degp_ref, hs_ref):
    dinv = _dinv_from_parts(degp_ref[...])
    h = jnp.dot(x_ref[...], w1_ref[...], preferred_element_type=jnp.float32)
    h = h * dinv
    hs_ref[0, :, :] = h[:, :128]
    hs_ref[1, :, :] = h[:, 128:]


_tc_a = pl.pallas_call(
    _tc_a_body,
    grid=(N // BN,),
    in_specs=[
        pl.BlockSpec((BN, D_IN), lambda i: (i, 0)),
        pl.BlockSpec((D_IN, D_H), lambda i: (0, 0)),
        pl.BlockSpec((2, BN, 128), lambda i: (0, i, 0)),
    ],
    out_specs=pl.BlockSpec((2, BN, 128), lambda i: (0, i, 0)),
    out_shape=jax.ShapeDtypeStruct((2, N, 128), jnp.float32),
)


def _tc_b_body(s1_ref, hs_ref, degp_ref, b1_ref, w2_ref, zp_ref):
    dinv = _dinv_from_parts(degp_ref[...])
    s = jnp.concatenate([s1_ref[0] + hs_ref[0], s1_ref[1] + hs_ref[1]], axis=1)
    pre = s * dinv + b1_ref[...]
    r = jnp.maximum(pre, 0.0)
    z = jnp.dot(r, w2_ref[...], preferred_element_type=jnp.float32)
    zp = z * dinv
    zp_ref[...] = jnp.concatenate(
        [zp, jnp.zeros((zp.shape[0], 128 - D_OUT), jnp.float32)], axis=1)


_tc_b = pl.pallas_call(
    _tc_b_body,
    grid=(N // BN,),
    in_specs=[
        pl.BlockSpec((2, BN, 128), lambda i: (0, i, 0)),
        pl.BlockSpec((2, BN, 128), lambda i: (0, i, 0)),
        pl.BlockSpec((2, BN, 128), lambda i: (0, i, 0)),
        pl.BlockSpec((1, D_H), lambda i: (0, 0)),
        pl.BlockSpec((D_H, D_OUT), lambda i: (0, 0)),
    ],
    out_specs=pl.BlockSpec((BN, 128), lambda i: (i, 0)),
    out_shape=jax.ShapeDtypeStruct((N, 128), jnp.float32),
)


def _tc_c_body(s2_ref, zp_ref, degp_ref, b2_ref, out_ref):
    dinv = _dinv_from_parts(degp_ref[...])
    pre = ((s2_ref[0, :, :D_OUT] + s2_ref[1, :, :D_OUT] + zp_ref[:, :D_OUT])
           * dinv + b2_ref[...])
    m = jnp.max(pre, axis=1, keepdims=True)
    e = pre - m
    out_ref[...] = e - jnp.log(jnp.sum(jnp.exp(e), axis=1, keepdims=True))


_tc_c = pl.pallas_call(
    _tc_c_body,
    grid=(N // BN,),
    in_specs=[
        pl.BlockSpec((2, BN, 128), lambda i: (0, i, 0)),
        pl.BlockSpec((BN, 128), lambda i: (i, 0)),
        pl.BlockSpec((2, BN, 128), lambda i: (0, i, 0)),
        pl.BlockSpec((1, D_OUT), lambda i: (0, 0)),
    ],
    out_specs=pl.BlockSpec((BN, D_OUT), lambda i: (i, 0)),
    out_shape=jax.ShapeDtypeStruct((N, D_OUT), jnp.float32),
)


@jax.jit
def kernel(x, edge_index, W1, b1, W2, b2):
    src = edge_index[0]
    dst = edge_index[1]
    pad = E_PAD - E
    src_p = jnp.concatenate([src, jnp.zeros((pad,), jnp.int32)])
    dst_p = jnp.concatenate([dst, jnp.full((pad,), N, jnp.int32)])
    src_r32 = src_p.reshape(NC * NS, E_PAD // (NC * NS) // CHUNK, CHUNK)
    dst_r32 = dst_p.reshape(NC * NS, E_PAD // (NC * NS) // CHUNK, CHUNK)
    dst_r16 = dst_p.reshape(NS, E_PAD // NS // CHUNK, CHUNK)
    srcs2 = jnp.concatenate([src_p, src_p + N]).reshape(
        NC * NS, E_PAD // NS // CHUNK, CHUNK)

    degp = _deg_call(dst_r32).reshape(2, ACC_ROWS, 128)
    hs = _tc_a(x, W1, degp)
    s1 = _agg256_call(srcs2, dst_r16, hs.reshape(2 * N, 128)).reshape(
        2, ACC_ROWS, 128)
    zp = _tc_b(s1, hs, degp, b1.reshape(1, D_H), W2)
    s2 = _agg64_call(src_r32, dst_r32, zp).reshape(2, ACC_ROWS, 128)
    return _tc_c(s2, zp, degp, b2.reshape(1, D_OUT))
